# baseline (speedup 1.0000x reference)
import functools

import jax
import jax.numpy as jnp
from jax import lax
from jax.experimental import pallas as pl
from jax.experimental.pallas import tpu as pltpu
from jax.experimental.pallas import tpu_sc as plsc

_D_MODEL = 4096
_N_EXPERTS = 64
_TOP_K = 8
_N_TOKENS = 32768
_BT = 1024

_NC = 2
_NS = 16
_L = 16
_NW = _NC * _NS
_UNROLL = 4


def _matmul_body(h_ref, w_ref, logits_ref):
    logits_ref[...] = lax.dot_general(
        h_ref[...], w_ref[...],
        dimension_numbers=(((1,), (1,)), ((), ())),
        preferred_element_type=jnp.float32,
    )


def _tc_logits(h, W):
    n_tokens = h.shape[0]
    return pl.pallas_call(
        _matmul_body,
        grid=(n_tokens // _BT,),
        in_specs=[
            pl.BlockSpec((_BT, _D_MODEL), lambda i: (i, 0)),
            pl.BlockSpec((_N_EXPERTS, _D_MODEL), lambda i: (0, 0)),
        ],
        out_specs=pl.BlockSpec((_BT, _N_EXPERTS), lambda i: (i, 0)),
        out_shape=jax.ShapeDtypeStruct((n_tokens, _N_EXPERTS), jnp.float32),
    )(h, W)


def _make_sc_router_body(tpw):
    n_groups = tpw // _L

    def _sc_router_body(logits_hbm, idx_hbm, wgt_hbm, lg_v, idx_v, wgt_v):
        wid = lax.axis_index("s") * _NC + lax.axis_index("c")
        pltpu.sync_copy(
            logits_hbm.at[pl.ds(wid * tpw * _N_EXPERTS, tpw * _N_EXPERTS)], lg_v
        )

        lane = lax.iota(jnp.int32, _L)
        neg_inf = jnp.full((_L,), -jnp.inf, dtype=jnp.float32)
        zero_i = jnp.zeros((_L,), dtype=jnp.int32)

        def insert(e, row_off, kv, ki):
            vid = jnp.full((_L,), e, jnp.int32)
            v = plsc.load_gather(lg_v, [row_off + e])
            for j in range(_TOP_K):
                gt = v > kv[j]
                kv[j], v = jnp.where(gt, v, kv[j]), jnp.where(gt, kv[j], v)
                ki[j], vid = (
                    jnp.where(gt, vid, ki[j]),
                    jnp.where(gt, ki[j], vid),
                )
            return kv, ki

        def finish(rows, kv, ki):
            es = [jnp.exp(kv[j] - kv[0]) for j in range(_TOP_K)]
            tot = es[0]
            for j in range(1, _TOP_K):
                tot = tot + es[j]
            inv = jnp.float32(1.0) / tot
            out_off = rows * _TOP_K
            for j in range(_TOP_K):
                plsc.store_scatter(idx_v, [out_off + j], ki[j])
                plsc.store_scatter(wgt_v, [out_off + j], es[j] * inv)

        def group_body(g, carry):
            rows_a = (2 * g) * _L + lane
            rows_b = (2 * g + 1) * _L + lane
            off_a = rows_a * _N_EXPERTS
            off_b = rows_b * _N_EXPERTS

            def expert_body(eb, kp):
                kva, kia, kvb, kib = (list(t) for t in kp)
                for u in range(_UNROLL):
                    e = eb * _UNROLL + u
                    kva, kia = insert(e, off_a, kva, kia)
                    kvb, kib = insert(e, off_b, kvb, kib)
                return (tuple(kva), tuple(kia), tuple(kvb), tuple(kib))

            init = (
                (neg_inf,) * _TOP_K, (zero_i,) * _TOP_K,
                (neg_inf,) * _TOP_K, (zero_i,) * _TOP_K,
            )
            kva, kia, kvb, kib = lax.fori_loop(
                0, _N_EXPERTS // _UNROLL, expert_body, init
            )
            finish(rows_a, list(kva), list(kia))
            finish(rows_b, list(kvb), list(kib))
            return carry

        lax.fori_loop(0, n_groups // 2, group_body, 0)

        pltpu.sync_copy(idx_v, idx_hbm.at[pl.ds(wid * tpw * _TOP_K, tpw * _TOP_K)])
        pltpu.sync_copy(wgt_v, wgt_hbm.at[pl.ds(wid * tpw * _TOP_K, tpw * _TOP_K)])

    return _sc_router_body


def _sc_route(logits_flat, n_tokens):
    tpw = n_tokens // _NW
    fn = pl.kernel(
        _make_sc_router_body(tpw),
        out_type=(
            jax.ShapeDtypeStruct((n_tokens * _TOP_K,), jnp.int32),
            jax.ShapeDtypeStruct((n_tokens * _TOP_K,), jnp.float32),
        ),
        mesh=plsc.VectorSubcoreMesh(core_axis_name="c", subcore_axis_name="s"),
        compiler_params=pltpu.CompilerParams(needs_layout_passes=False),
        scratch_types=[
            pltpu.VMEM((tpw * _N_EXPERTS,), jnp.float32),
            pltpu.VMEM((tpw * _TOP_K,), jnp.int32),
            pltpu.VMEM((tpw * _TOP_K,), jnp.float32),
        ],
    )
    return fn(logits_flat)


@jax.jit
def kernel(h, W):
    n_tokens = h.shape[0]
    logits = _tc_logits(h, W)
    topi, wgt = _sc_route(logits.reshape(-1), n_tokens)
    return (
        topi.reshape(n_tokens, _TOP_K),
        wgt.reshape(n_tokens, _TOP_K),
        logits,
    )


# speedup vs baseline: 1.0912x; 1.0912x over previous
import functools

import jax
import jax.numpy as jnp
from jax import lax
from jax.experimental import pallas as pl
from jax.experimental.pallas import tpu as pltpu
from jax.experimental.pallas import tpu_sc as plsc

_D_MODEL = 4096
_N_EXPERTS = 64
_TOP_K = 8
_N_TOKENS = 32768
_BT = 1024

_NC = 2
_NS = 16
_L = 16
_NW = _NC * _NS


def _matmul_body(h_ref, w_ref, logits_ref):
    logits_ref[...] = lax.dot_general(
        h_ref[...], w_ref[...],
        dimension_numbers=(((1,), (1,)), ((), ())),
        preferred_element_type=jnp.float32,
    )


def _tc_logits(h, W):
    n_tokens = h.shape[0]
    return pl.pallas_call(
        _matmul_body,
        grid=(n_tokens // _BT,),
        in_specs=[
            pl.BlockSpec((_BT, _D_MODEL), lambda i: (i, 0)),
            pl.BlockSpec((_N_EXPERTS, _D_MODEL), lambda i: (0, 0)),
        ],
        out_specs=pl.BlockSpec((_BT, _N_EXPERTS), lambda i: (i, 0)),
        out_shape=jax.ShapeDtypeStruct((n_tokens, _N_EXPERTS), jnp.float32),
    )(h, W)


def _make_sc_router_body(tpw):
    n_groups = tpw // _L

    def _sc_router_body(logits_hbm, idx_hbm, wgt_hbm, lg_v, idx_v, wgt_v):
        wid = lax.axis_index("s") * _NC + lax.axis_index("c")
        pltpu.sync_copy(
            logits_hbm.at[pl.ds(wid * tpw * _N_EXPERTS, tpw * _N_EXPERTS)], lg_v
        )

        lane = lax.iota(jnp.int32, _L)
        neg_inf = jnp.full((_L,), -jnp.inf, dtype=jnp.float32)
        mask_hi = jnp.full((_L,), ~0x3F, dtype=jnp.int32)
        lo_mask = jnp.full((_L,), 0x3F, dtype=jnp.int32)
        id_max = jnp.full((_L,), _N_EXPERTS - 1, dtype=jnp.int32)

        def group_body(g, carry):
            rows = g * _L + lane
            row_off = rows * _N_EXPERTS

            kv = [neg_inf] * _TOP_K
            for e in range(_N_EXPERTS):
                v = plsc.load_gather(lg_v, [row_off + e])
                vb = plsc.bitcast(v, jnp.int32)
                idb = jnp.full((_L,), _N_EXPERTS - 1 - e, dtype=jnp.int32)
                key = plsc.bitcast((vb & mask_hi) | idb, jnp.float32)
                for j in range(_TOP_K):
                    hi = jnp.maximum(kv[j], key)
                    key = jnp.minimum(kv[j], key)
                    kv[j] = hi

            ki = []
            vs = []
            for j in range(_TOP_K):
                e_id = id_max - (plsc.bitcast(kv[j], jnp.int32) & lo_mask)
                ki.append(e_id)
                vs.append(plsc.load_gather(lg_v, [row_off + e_id]))

            es = [jnp.exp(vs[j] - vs[0]) for j in range(_TOP_K)]
            tot = es[0]
            for j in range(1, _TOP_K):
                tot = tot + es[j]
            inv = jnp.float32(1.0) / tot

            out_off = rows * _TOP_K
            for j in range(_TOP_K):
                plsc.store_scatter(idx_v, [out_off + j], ki[j])
                plsc.store_scatter(wgt_v, [out_off + j], es[j] * inv)
            return carry

        lax.fori_loop(0, n_groups, group_body, 0)

        pltpu.sync_copy(idx_v, idx_hbm.at[pl.ds(wid * tpw * _TOP_K, tpw * _TOP_K)])
        pltpu.sync_copy(wgt_v, wgt_hbm.at[pl.ds(wid * tpw * _TOP_K, tpw * _TOP_K)])

    return _sc_router_body


def _sc_route(logits_flat, n_tokens):
    tpw = n_tokens // _NW
    fn = pl.kernel(
        _make_sc_router_body(tpw),
        out_type=(
            jax.ShapeDtypeStruct((n_tokens * _TOP_K,), jnp.int32),
            jax.ShapeDtypeStruct((n_tokens * _TOP_K,), jnp.float32),
        ),
        mesh=plsc.VectorSubcoreMesh(core_axis_name="c", subcore_axis_name="s"),
        compiler_params=pltpu.CompilerParams(needs_layout_passes=False),
        scratch_types=[
            pltpu.VMEM((tpw * _N_EXPERTS,), jnp.float32),
            pltpu.VMEM((tpw * _TOP_K,), jnp.int32),
            pltpu.VMEM((tpw * _TOP_K,), jnp.float32),
        ],
    )
    return fn(logits_flat)


@jax.jit
def kernel(h, W):
    n_tokens = h.shape[0]
    logits = _tc_logits(h, W)
    topi, wgt = _sc_route(logits.reshape(-1), n_tokens)
    return (
        topi.reshape(n_tokens, _TOP_K),
        wgt.reshape(n_tokens, _TOP_K),
        logits,
    )
